# baseline (speedup 1.0000x reference)
import functools

import jax
import jax.numpy as jnp
from jax import lax
from jax.experimental import pallas as pl
from jax.experimental.pallas import tpu as pltpu
from jax.experimental.pallas import tpu_sc as plsc

_N = 10000
_E = 320000
_D = 128
_OUT = 64

_RANGE = 320
_NP = 10240

_BLK = 5000
_TILES = _BLK // 8

_BLKF = 2000
_NBLKF = _N // _BLKF


def _sc_mesh():
    return plsc.VectorSubcoreMesh(core_axis_name="c", subcore_axis_name="s")


def _zero_rows_buf(rows):
    def _zrow(i, carry):
        for j in range(_D // 16):
            rows[i, pl.ds(j * 16, 16)] = jnp.zeros((16,), jnp.float32)
        return carry
    lax.fori_loop(0, 128, _zrow, 0)


def _zero_acc_range(rows, acc, lo):
    pltpu.sync_copy(rows, acc.at[pl.ds(lo, 128)])
    pltpu.sync_copy(rows, acc.at[pl.ds(lo + 128, 128)])
    pltpu.sync_copy(rows.at[pl.ds(0, 64)], acc.at[pl.ds(lo + 256, 64)])


def _copy_acc_range_out(rows, acc, lo, out):
    pltpu.sync_copy(acc.at[pl.ds(lo, 128)], rows)
    pltpu.sync_copy(rows, out.at[pl.ds(lo, 128)])
    pltpu.sync_copy(acc.at[pl.ds(lo + 128, 128)], rows)
    pltpu.sync_copy(rows, out.at[pl.ds(lo + 128, 128)])
    pltpu.sync_copy(acc.at[pl.ds(lo + 256, 64)], rows.at[pl.ds(0, 64)])
    pltpu.sync_copy(rows.at[pl.ds(0, 64)], out.at[pl.ds(lo + 256, 64)])


def _segsum(x, srcp, dstp, offs):
    @functools.partial(
        pl.kernel,
        out_type=jax.ShapeDtypeStruct((_NP, _D), jnp.float32),
        mesh=_sc_mesh(),
        scratch_types=[
            pltpu.VMEM((48,), jnp.int32),
            pltpu.VMEM((1, 128), jnp.int32),
            pltpu.VMEM((1, 128), jnp.int32),
            pltpu.VMEM((128, _D), jnp.float32),
            pltpu.VMEM_SHARED((_NP, _D), jnp.float32),
            pltpu.SemaphoreType.DMA,
        ],
    )
    def k(x_hbm, src_hbm, dst_hbm, off_hbm, agg_hbm,
          obuf, sidx, didx, rows, acc, sem):
        c = lax.axis_index("c")
        s = lax.axis_index("s")
        w = c * 16 + s
        lo = w * _RANGE

        _zero_rows_buf(rows)
        _zero_acc_range(rows, acc, lo)

        pltpu.sync_copy(off_hbm.at[pl.ds(0, 48)], obuf)
        ov = obuf[pl.ds(w, 16)]
        o0 = ov[0]
        o1 = ov[1]
        start = (o0 // 128) * 128
        n = (o1 - start + 127) // 128

        iota = lax.iota(jnp.int32, 16)
        dumv = jnp.full((16,), _N, jnp.int32)
        zv = jnp.zeros((16,), jnp.int32)

        def _blk(bi, carry):
            gbase = start + bi * 128
            pltpu.sync_copy(src_hbm.at[pl.ds(gbase, 128)], sidx.at[0])
            pltpu.sync_copy(dst_hbm.at[pl.ds(gbase, 128)], didx.at[0])
            for j in range(8):
                gi = gbase + j * 16 + iota
                keep = ((gi - o0) >= 0) & ((gi - o1) < 0)
                dv = didx[0, pl.ds(j * 16, 16)]
                didx[0, pl.ds(j * 16, 16)] = jnp.where(keep, dv, dumv)
                sv = sidx[0, pl.ds(j * 16, 16)]
                sidx[0, pl.ds(j * 16, 16)] = jnp.where(keep, sv, zv)
            pltpu.async_copy(x_hbm.at[sidx.at[0]], rows, sem).wait()
            pltpu.sync_copy(rows, acc.at[didx.at[0]], add=True)
            return carry
        lax.fori_loop(0, n, _blk, 0)

        _copy_acc_range_out(rows, acc, lo, agg_hbm)

    return k(x, srcp, dstp, offs)


def _fold8(a):
    a4 = a[0:4, :] + a[4:8, :]
    a2 = a4[0:2, :] + a4[2:4, :]
    return a2[0:1, :] + a2[1:2, :]


def _mlp_body(x_ref, agg_ref, w1, b1, w2, b2, w3, b3, h_ref, sum_ref):
    i = pl.program_id(0)
    h0 = x_ref[...] + agg_ref[...]
    h = jnp.maximum(jnp.dot(h0, w1[...], preferred_element_type=jnp.float32)
                    + b1[...], 0.0)
    h = jnp.maximum(jnp.dot(h, w2[...], preferred_element_type=jnp.float32)
                    + b2[...], 0.0)
    h = jnp.dot(h, w3[...], preferred_element_type=jnp.float32) + b3[...]
    h_ref[...] = h

    def _acc(t, a):
        return a + h_ref[pl.ds(t * 8, 8), :]
    part = _fold8(lax.fori_loop(0, _TILES, _acc,
                                jnp.zeros((8, _D), jnp.float32)))
    part = jnp.broadcast_to(part, sum_ref.shape)

    @pl.when(i == 0)
    def _():
        sum_ref[...] = part

    @pl.when(i != 0)
    def _():
        sum_ref[...] += part


def _mlp_stats(x, agg, w1, b1, w2, b2, w3, b3):
    return pl.pallas_call(
        _mlp_body,
        grid=(2,),
        in_specs=[
            pl.BlockSpec((_BLK, _D), lambda i: (i, 0)),
            pl.BlockSpec((_BLK, _D), lambda i: (i, 0)),
            pl.BlockSpec((_D, _D), lambda i: (0, 0)),
            pl.BlockSpec((1, _D), lambda i: (0, 0)),
            pl.BlockSpec((_D, _D), lambda i: (0, 0)),
            pl.BlockSpec((1, _D), lambda i: (0, 0)),
            pl.BlockSpec((_D, _D), lambda i: (0, 0)),
            pl.BlockSpec((1, _D), lambda i: (0, 0)),
        ],
        out_specs=[
            pl.BlockSpec((_BLK, _D), lambda i: (i, 0)),
            pl.BlockSpec((8, _D), lambda i: (0, 0)),
        ],
        out_shape=[
            jax.ShapeDtypeStruct((_N, _D), jnp.float32),
            jax.ShapeDtypeStruct((8, _D), jnp.float32),
        ],
    )(x, agg, w1, b1, w2, b2, w3, b3)


def _sq_body(h_ref, sum_ref, sq_ref):
    i = pl.program_id(0)
    mu = sum_ref[0:1, :] * (1.0 / _N)

    def _acc(t, a):
        ct = h_ref[pl.ds(t * 8, 8), :] - mu
        return a + ct * ct
    part = _fold8(lax.fori_loop(0, _TILES, _acc,
                                jnp.zeros((8, _D), jnp.float32)))
    part = jnp.broadcast_to(part, sq_ref.shape)

    @pl.when(i == 0)
    def _():
        sq_ref[...] = part

    @pl.when(i != 0)
    def _():
        sq_ref[...] += part


def _sq_stats(h, ssum):
    return pl.pallas_call(
        _sq_body,
        grid=(2,),
        in_specs=[
            pl.BlockSpec((_BLK, _D), lambda i: (i, 0)),
            pl.BlockSpec((8, _D), lambda i: (0, 0)),
        ],
        out_specs=pl.BlockSpec((8, _D), lambda i: (0, 0)),
        out_shape=jax.ShapeDtypeStruct((8, _D), jnp.float32),
    )(h, ssum)


def _bn_body(relu, h_ref, sum_ref, sq_ref, g_ref, b_ref, o_ref):
    mu = sum_ref[0:1, :] * (1.0 / _N)
    var = sq_ref[0:1, :] * (1.0 / _N)
    y = (h_ref[...] - mu) * lax.rsqrt(var + 1e-5) * g_ref[...] + b_ref[...]
    if relu:
        y = jnp.maximum(y, 0.0)
    o_ref[...] = y


def _bn_apply(h, ssum, ssq, g, b, relu):
    return pl.pallas_call(
        functools.partial(_bn_body, relu),
        grid=(2,),
        in_specs=[
            pl.BlockSpec((_BLK, _D), lambda i: (i, 0)),
            pl.BlockSpec((8, _D), lambda i: (0, 0)),
            pl.BlockSpec((8, _D), lambda i: (0, 0)),
            pl.BlockSpec((1, _D), lambda i: (0, 0)),
            pl.BlockSpec((1, _D), lambda i: (0, 0)),
        ],
        out_specs=pl.BlockSpec((_BLK, _D), lambda i: (i, 0)),
        out_shape=jax.ShapeDtypeStruct((_N, _D), jnp.float32),
    )(h, ssum, ssq, g, b)


def _final_body(h1_ref, h2_ref, h3_ref, wa, wb, wc, lb, cw, cb,
                h_ref, logit_ref):
    h = (jnp.dot(h1_ref[...], wa[0], preferred_element_type=jnp.float32)
         + jnp.dot(h2_ref[...], wb[0], preferred_element_type=jnp.float32)
         + jnp.dot(h3_ref[...], wc[0], preferred_element_type=jnp.float32)
         + lb[...])
    h_ref[...] = h
    logit_ref[...] = jnp.dot(h, cw[...], preferred_element_type=jnp.float32) \
        + cb[...]


def _final(h1, h2, h3, lin_W, lin_b, cls_W, cls_b):
    lw3 = lin_W.reshape(3, _D, _D)
    return pl.pallas_call(
        _final_body,
        grid=(_NBLKF,),
        in_specs=[
            pl.BlockSpec((_BLKF, _D), lambda i: (i, 0)),
            pl.BlockSpec((_BLKF, _D), lambda i: (i, 0)),
            pl.BlockSpec((_BLKF, _D), lambda i: (i, 0)),
            pl.BlockSpec((1, _D, _D), lambda i: (0, 0, 0)),
            pl.BlockSpec((1, _D, _D), lambda i: (1, 0, 0)),
            pl.BlockSpec((1, _D, _D), lambda i: (2, 0, 0)),
            pl.BlockSpec((1, _D), lambda i: (0, 0)),
            pl.BlockSpec((_D, _OUT), lambda i: (0, 0)),
            pl.BlockSpec((1, _OUT), lambda i: (0, 0)),
        ],
        out_specs=[
            pl.BlockSpec((_BLKF, _D), lambda i: (i, 0)),
            pl.BlockSpec((_BLKF, _OUT), lambda i: (i, 0)),
        ],
        out_shape=[
            jax.ShapeDtypeStruct((_N, _D), jnp.float32),
            jax.ShapeDtypeStruct((_N, _OUT), jnp.float32),
        ],
    )(h1, h2, h3, lw3, lw3, lw3, lin_b, cls_W, cls_b)


def kernel(x, edge_index,
           c1_W1, c1_b1, c1_W2, c1_b2, c1_W3, c1_b3, bn1_g, bn1_b,
           c2_W1, c2_b1, c2_W2, c2_b2, c2_W3, c2_b3, bn2_g, bn2_b,
           c3_W1, c3_b1, c3_W2, c3_b2, c3_W3, c3_b3, bn3_g, bn3_b,
           lin_W, lin_b, cls_W, cls_b):
    src = edge_index[0]
    dst = edge_index[1]
    bucket = dst // _RANGE
    perm = jnp.argsort(bucket, stable=True)
    srcp = src[perm]
    dstp = dst[perm]
    offs = jnp.searchsorted(bucket[perm], jnp.arange(33, dtype=jnp.int32),
                            side="left").astype(jnp.int32)
    offs = jnp.concatenate([offs, jnp.full((15,), _E, jnp.int32)])

    r2 = lambda v: v.reshape(1, -1)
    layers = [
        (c1_W1, r2(c1_b1), c1_W2, r2(c1_b2), c1_W3, r2(c1_b3),
         r2(bn1_g), r2(bn1_b), True),
        (c2_W1, r2(c2_b1), c2_W2, r2(c2_b2), c2_W3, r2(c2_b3),
         r2(bn2_g), r2(bn2_b), True),
        (c3_W1, r2(c3_b1), c3_W2, r2(c3_b2), c3_W3, r2(c3_b3),
         r2(bn3_g), r2(bn3_b), False),
    ]

    hs = []
    h = x
    for (w1, b1, w2, b2, w3, b3, g, b, relu) in layers:
        agg = _segsum(h, srcp, dstp, offs)
        h_pre, ssum = _mlp_stats(h, agg, w1, b1, w2, b2, w3, b3)
        ssq = _sq_stats(h_pre, ssum)
        h = _bn_apply(h_pre, ssum, ssq, g, b, relu)
        hs.append(h)

    h_out, logits = _final(hs[0], hs[1], hs[2], lin_W, r2(lin_b),
                           cls_W, r2(cls_b))
    return (logits, h_out)


# speedup vs baseline: 1.1871x; 1.1871x over previous
import functools

import jax
import jax.numpy as jnp
from jax import lax
from jax.experimental import pallas as pl
from jax.experimental.pallas import tpu as pltpu
from jax.experimental.pallas import tpu_sc as plsc

_N = 10000
_E = 320000
_D = 128
_OUT = 64

_RANGE = 320
_NP = 10240

_BLK = 5000
_TILES = _BLK // 8

_BLKF = 2000
_NBLKF = _N // _BLKF


def _sc_mesh():
    return plsc.VectorSubcoreMesh(core_axis_name="c", subcore_axis_name="s")


def _zero_rows_buf(rows):
    def _zrow(i, carry):
        for j in range(_D // 16):
            rows[i, pl.ds(j * 16, 16)] = jnp.zeros((16,), jnp.float32)
        return carry
    lax.fori_loop(0, 128, _zrow, 0)


def _zero_acc_range(rows, acc, lo):
    pltpu.sync_copy(rows, acc.at[pl.ds(lo, 128)])
    pltpu.sync_copy(rows, acc.at[pl.ds(lo + 128, 128)])
    pltpu.sync_copy(rows.at[pl.ds(0, 64)], acc.at[pl.ds(lo + 256, 64)])


def _copy_acc_range_out(rows, acc, lo, out):
    pltpu.sync_copy(acc.at[pl.ds(lo, 128)], rows)
    pltpu.sync_copy(rows, out.at[pl.ds(lo, 128)])
    pltpu.sync_copy(acc.at[pl.ds(lo + 128, 128)], rows)
    pltpu.sync_copy(rows, out.at[pl.ds(lo + 128, 128)])
    pltpu.sync_copy(acc.at[pl.ds(lo + 256, 64)], rows.at[pl.ds(0, 64)])
    pltpu.sync_copy(rows.at[pl.ds(0, 64)], out.at[pl.ds(lo + 256, 64)])


def _segsum(x, srcp, dstp, offs):
    @functools.partial(
        pl.kernel,
        out_type=jax.ShapeDtypeStruct((_NP, _D), jnp.float32),
        mesh=_sc_mesh(),
        scratch_types=[
            pltpu.VMEM((48,), jnp.int32),
            pltpu.VMEM((2, 128), jnp.int32),
            pltpu.VMEM((2, 128), jnp.int32),
            pltpu.VMEM((128, _D), jnp.float32),
            pltpu.VMEM((128, _D), jnp.float32),
            pltpu.VMEM_SHARED((_NP, _D), jnp.float32),
            pltpu.SemaphoreType.DMA,
            pltpu.SemaphoreType.DMA,
        ],
    )
    def k(x_hbm, src_hbm, dst_hbm, off_hbm, agg_hbm,
          obuf, sidx, didx, rows0, rows1, acc, sem0, sem1):
        c = lax.axis_index("c")
        s = lax.axis_index("s")
        w = c * 16 + s
        lo = w * _RANGE

        _zero_rows_buf(rows0)
        _zero_acc_range(rows0, acc, lo)

        pltpu.sync_copy(off_hbm.at[pl.ds(0, 48)], obuf)
        ov = obuf[pl.ds(w, 16)]
        o0 = ov[0]
        o1 = ov[1]
        start = (o0 // 128) * 128
        n = (o1 - start + 127) // 128

        iota = lax.iota(jnp.int32, 16)
        dumv = jnp.full((16,), _N, jnp.int32)
        zv = jnp.zeros((16,), jnp.int32)
        slots = ((rows0, sem0), (rows1, sem1))

        def _stage(bi, p):
            rows_p, sem_p = slots[p]
            gbase = start + bi * 128
            pltpu.sync_copy(src_hbm.at[pl.ds(gbase, 128)], sidx.at[p])
            pltpu.sync_copy(dst_hbm.at[pl.ds(gbase, 128)], didx.at[p])
            for j in range(8):
                gi = gbase + j * 16 + iota
                keep = ((gi - o0) >= 0) & ((gi - o1) < 0)
                dv = didx[p, pl.ds(j * 16, 16)]
                didx[p, pl.ds(j * 16, 16)] = jnp.where(keep, dv, dumv)
                sv = sidx[p, pl.ds(j * 16, 16)]
                sidx[p, pl.ds(j * 16, 16)] = jnp.where(keep, sv, zv)
            pltpu.async_copy(x_hbm.at[sidx.at[p]], rows_p, sem_p)

        def _drain(p):
            rows_p, sem_p = slots[p]
            pltpu.make_async_copy(x_hbm.at[pl.ds(0, 128)], rows_p,
                                  sem_p).wait()

        @pl.when(n > 0)
        def _():
            _stage(0, 0)

        def _pair(g, carry):
            for p in (0, 1):
                bi = g * 2 + p

                @pl.when(bi < n)
                def _():
                    @pl.when(bi + 1 < n)
                    def _():
                        _stage(bi + 1, 1 - p)
                    _drain(p)
                    rows_p, _sem = slots[p]
                    pltpu.sync_copy(rows_p, acc.at[didx.at[p]], add=True)
            return carry
        lax.fori_loop(0, (n + 1) // 2, _pair, 0)

        _copy_acc_range_out(rows0, acc, lo, agg_hbm)

    return k(x, srcp, dstp, offs)


def _fold8(a):
    a4 = a[0:4, :] + a[4:8, :]
    a2 = a4[0:2, :] + a4[2:4, :]
    return a2[0:1, :] + a2[1:2, :]


def _mlp_body(x_ref, agg_ref, w1, b1, w2, b2, w3, b3, h_ref, sum_ref):
    i = pl.program_id(0)
    h0 = x_ref[...] + agg_ref[...]
    h = jnp.maximum(jnp.dot(h0, w1[...], preferred_element_type=jnp.float32)
                    + b1[...], 0.0)
    h = jnp.maximum(jnp.dot(h, w2[...], preferred_element_type=jnp.float32)
                    + b2[...], 0.0)
    h = jnp.dot(h, w3[...], preferred_element_type=jnp.float32) + b3[...]
    h_ref[...] = h

    def _acc(t, a):
        return a + h_ref[pl.ds(t * 8, 8), :]
    part = _fold8(lax.fori_loop(0, _TILES, _acc,
                                jnp.zeros((8, _D), jnp.float32)))
    part = jnp.broadcast_to(part, sum_ref.shape)

    @pl.when(i == 0)
    def _():
        sum_ref[...] = part

    @pl.when(i != 0)
    def _():
        sum_ref[...] += part


def _mlp_stats(x, agg, w1, b1, w2, b2, w3, b3):
    return pl.pallas_call(
        _mlp_body,
        grid=(2,),
        in_specs=[
            pl.BlockSpec((_BLK, _D), lambda i: (i, 0)),
            pl.BlockSpec((_BLK, _D), lambda i: (i, 0)),
            pl.BlockSpec((_D, _D), lambda i: (0, 0)),
            pl.BlockSpec((1, _D), lambda i: (0, 0)),
            pl.BlockSpec((_D, _D), lambda i: (0, 0)),
            pl.BlockSpec((1, _D), lambda i: (0, 0)),
            pl.BlockSpec((_D, _D), lambda i: (0, 0)),
            pl.BlockSpec((1, _D), lambda i: (0, 0)),
        ],
        out_specs=[
            pl.BlockSpec((_BLK, _D), lambda i: (i, 0)),
            pl.BlockSpec((8, _D), lambda i: (0, 0)),
        ],
        out_shape=[
            jax.ShapeDtypeStruct((_N, _D), jnp.float32),
            jax.ShapeDtypeStruct((8, _D), jnp.float32),
        ],
    )(x, agg, w1, b1, w2, b2, w3, b3)


def _sq_body(h_ref, sum_ref, sq_ref):
    i = pl.program_id(0)
    mu = sum_ref[0:1, :] * (1.0 / _N)

    def _acc(t, a):
        ct = h_ref[pl.ds(t * 8, 8), :] - mu
        return a + ct * ct
    part = _fold8(lax.fori_loop(0, _TILES, _acc,
                                jnp.zeros((8, _D), jnp.float32)))
    part = jnp.broadcast_to(part, sq_ref.shape)

    @pl.when(i == 0)
    def _():
        sq_ref[...] = part

    @pl.when(i != 0)
    def _():
        sq_ref[...] += part


def _sq_stats(h, ssum):
    return pl.pallas_call(
        _sq_body,
        grid=(2,),
        in_specs=[
            pl.BlockSpec((_BLK, _D), lambda i: (i, 0)),
            pl.BlockSpec((8, _D), lambda i: (0, 0)),
        ],
        out_specs=pl.BlockSpec((8, _D), lambda i: (0, 0)),
        out_shape=jax.ShapeDtypeStruct((8, _D), jnp.float32),
    )(h, ssum)


def _bn_body(relu, h_ref, sum_ref, sq_ref, g_ref, b_ref, o_ref):
    mu = sum_ref[0:1, :] * (1.0 / _N)
    var = sq_ref[0:1, :] * (1.0 / _N)
    y = (h_ref[...] - mu) * lax.rsqrt(var + 1e-5) * g_ref[...] + b_ref[...]
    if relu:
        y = jnp.maximum(y, 0.0)
    o_ref[...] = y


def _bn_apply(h, ssum, ssq, g, b, relu):
    return pl.pallas_call(
        functools.partial(_bn_body, relu),
        grid=(2,),
        in_specs=[
            pl.BlockSpec((_BLK, _D), lambda i: (i, 0)),
            pl.BlockSpec((8, _D), lambda i: (0, 0)),
            pl.BlockSpec((8, _D), lambda i: (0, 0)),
            pl.BlockSpec((1, _D), lambda i: (0, 0)),
            pl.BlockSpec((1, _D), lambda i: (0, 0)),
        ],
        out_specs=pl.BlockSpec((_BLK, _D), lambda i: (i, 0)),
        out_shape=jax.ShapeDtypeStruct((_N, _D), jnp.float32),
    )(h, ssum, ssq, g, b)


def _final_body(h1_ref, h2_ref, h3_ref, wa, wb, wc, lb, cw, cb,
                h_ref, logit_ref):
    h = (jnp.dot(h1_ref[...], wa[0], preferred_element_type=jnp.float32)
         + jnp.dot(h2_ref[...], wb[0], preferred_element_type=jnp.float32)
         + jnp.dot(h3_ref[...], wc[0], preferred_element_type=jnp.float32)
         + lb[...])
    h_ref[...] = h
    logit_ref[...] = jnp.dot(h, cw[...], preferred_element_type=jnp.float32) \
        + cb[...]


def _final(h1, h2, h3, lin_W, lin_b, cls_W, cls_b):
    lw3 = lin_W.reshape(3, _D, _D)
    return pl.pallas_call(
        _final_body,
        grid=(_NBLKF,),
        in_specs=[
            pl.BlockSpec((_BLKF, _D), lambda i: (i, 0)),
            pl.BlockSpec((_BLKF, _D), lambda i: (i, 0)),
            pl.BlockSpec((_BLKF, _D), lambda i: (i, 0)),
            pl.BlockSpec((1, _D, _D), lambda i: (0, 0, 0)),
            pl.BlockSpec((1, _D, _D), lambda i: (1, 0, 0)),
            pl.BlockSpec((1, _D, _D), lambda i: (2, 0, 0)),
            pl.BlockSpec((1, _D), lambda i: (0, 0)),
            pl.BlockSpec((_D, _OUT), lambda i: (0, 0)),
            pl.BlockSpec((1, _OUT), lambda i: (0, 0)),
        ],
        out_specs=[
            pl.BlockSpec((_BLKF, _D), lambda i: (i, 0)),
            pl.BlockSpec((_BLKF, _OUT), lambda i: (i, 0)),
        ],
        out_shape=[
            jax.ShapeDtypeStruct((_N, _D), jnp.float32),
            jax.ShapeDtypeStruct((_N, _OUT), jnp.float32),
        ],
    )(h1, h2, h3, lw3, lw3, lw3, lin_b, cls_W, cls_b)


def kernel(x, edge_index,
           c1_W1, c1_b1, c1_W2, c1_b2, c1_W3, c1_b3, bn1_g, bn1_b,
           c2_W1, c2_b1, c2_W2, c2_b2, c2_W3, c2_b3, bn2_g, bn2_b,
           c3_W1, c3_b1, c3_W2, c3_b2, c3_W3, c3_b3, bn3_g, bn3_b,
           lin_W, lin_b, cls_W, cls_b):
    src = edge_index[0]
    dst = edge_index[1]
    bucket = dst // _RANGE
    perm = jnp.argsort(bucket, stable=True)
    srcp = src[perm]
    dstp = dst[perm]
    offs = jnp.searchsorted(bucket[perm], jnp.arange(33, dtype=jnp.int32),
                            side="left").astype(jnp.int32)
    offs = jnp.concatenate([offs, jnp.full((15,), _E, jnp.int32)])

    r2 = lambda v: v.reshape(1, -1)
    layers = [
        (c1_W1, r2(c1_b1), c1_W2, r2(c1_b2), c1_W3, r2(c1_b3),
         r2(bn1_g), r2(bn1_b), True),
        (c2_W1, r2(c2_b1), c2_W2, r2(c2_b2), c2_W3, r2(c2_b3),
         r2(bn2_g), r2(bn2_b), True),
        (c3_W1, r2(c3_b1), c3_W2, r2(c3_b2), c3_W3, r2(c3_b3),
         r2(bn3_g), r2(bn3_b), False),
    ]

    hs = []
    h = x
    for (w1, b1, w2, b2, w3, b3, g, b, relu) in layers:
        agg = _segsum(h, srcp, dstp, offs)
        h_pre, ssum = _mlp_stats(h, agg, w1, b1, w2, b2, w3, b3)
        ssq = _sq_stats(h_pre, ssum)
        h = _bn_apply(h_pre, ssum, ssq, g, b, relu)
        hs.append(h)

    h_out, logits = _final(hs[0], hs[1], hs[2], lin_W, r2(lin_b),
                           cls_W, r2(cls_b))
    return (logits, h_out)
